# Initial kernel scaffold; baseline (speedup 1.0000x reference)
import jax
import jax.numpy as jnp
from jax.experimental import pallas as pl


def kernel(times_list, node_pairs, x0, v, beta):
    raise NotImplementedError("write your pallas kernel here")


# speedup vs baseline: 10.7500x; 10.7500x over previous
import functools

import jax
import jax.numpy as jnp
from jax import lax
from jax.experimental import pallas as pl
from jax.experimental.pallas import tpu as pltpu
from jax.experimental.pallas import tpu_sc as plsc

_N = 100000
_D = 16
_BINS = 10
_P = 262144
_BIN_W = 1.0 / float(_BINS)

_NC, _NS, _L = 2, 16, 16
_NW = _NC * _NS
_EW = _P // _NW
_E = 1024
_NCHUNK = _EW // _E
_G = _E // _L

_C = 12800


def _build_a_body(x0_ref, v_ref, a_ref):
    acc = x0_ref[0, :]
    for b in range(_BINS):
        a_ref[b, :] = acc
        acc = acc + jnp.float32(_BIN_W) * v_ref[b, :]


_build_a = pl.pallas_call(
    _build_a_body,
    grid=(_N * _D // _C,),
    in_specs=[
        pl.BlockSpec((1, _C), lambda i: (0, i)),
        pl.BlockSpec((_BINS, _C), lambda i: (0, i)),
    ],
    out_specs=pl.BlockSpec((_BINS, _C), lambda i: (0, i)),
    out_shape=jax.ShapeDtypeStruct((_BINS, _N * _D), jnp.float32),
)

_mesh = plsc.VectorSubcoreMesh(
    core_axis_name="c", subcore_axis_name="s", num_cores=_NC, num_subcores=_NS
)


@functools.partial(
    pl.kernel,
    mesh=_mesh,
    out_type=jax.ShapeDtypeStruct((_P,), jnp.float32),
    compiler_params=pltpu.CompilerParams(
        needs_layout_passes=False, use_tc_tiling_on_sc=False
    ),
    scratch_types=[
        pltpu.VMEM((_E,), jnp.float32),
        pltpu.VMEM((_E,), jnp.int32),
        pltpu.VMEM((_E,), jnp.int32),
        pltpu.VMEM((_E,), jnp.int32),
        pltpu.VMEM((_E,), jnp.int32),
        pltpu.VMEM((_E,), jnp.float32),
        pltpu.VMEM((_E, _D), jnp.float32),
        pltpu.VMEM((_E, _D), jnp.float32),
        pltpu.VMEM((_E, _D), jnp.float32),
        pltpu.VMEM((_E, _D), jnp.float32),
        pltpu.VMEM((_E,), jnp.float32),
        pltpu.VMEM((_E,), jnp.float32),
        pltpu.VMEM((_E,), jnp.float32),
        pltpu.SemaphoreType.DMA,
    ],
)
def _sc_intensity(t_hbm, i_hbm, j_hbm, a_hbm, v_hbm, beta_hbm, out_hbm,
                  t_v, i_v, j_v, gi_v, gj_v, r_v,
                  ai_v, aj_v, vi_v, vj_v, bi_v, bj_v, o_v, sem):
    wid = lax.axis_index("s") * _NC + lax.axis_index("c")
    base = wid * _EW

    def chunk_body(c, carry):
        off = base + c * _E
        pltpu.sync_copy(t_hbm.at[pl.ds(off, _E)], t_v)
        pltpu.sync_copy(i_hbm.at[pl.ds(off, _E)], i_v)
        pltpu.sync_copy(j_hbm.at[pl.ds(off, _E)], j_v)

        def idx_body(g, carry2):
            s = pl.ds(g * _L, _L)
            t = t_v[s]
            b = (t / jnp.float32(_BIN_W)).astype(jnp.int32)
            b = jnp.where(b == _BINS, _BINS - 1, b)
            gi_v[s] = b * _N + i_v[s]
            gj_v[s] = b * _N + j_v[s]
            r_v[s] = jnp.remainder(t, jnp.float32(_BIN_W))
            return carry2

        lax.fori_loop(0, _G, idx_body, 0)

        cps = [
            pltpu.async_copy(a_hbm.at[gi_v], ai_v, sem),
            pltpu.async_copy(a_hbm.at[gj_v], aj_v, sem),
            pltpu.async_copy(v_hbm.at[gi_v], vi_v, sem),
            pltpu.async_copy(v_hbm.at[gj_v], vj_v, sem),
            pltpu.async_copy(beta_hbm.at[i_v], bi_v, sem),
            pltpu.async_copy(beta_hbm.at[j_v], bj_v, sem),
        ]
        for cp in cps:
            cp.wait()

        def grp_body(g, carry2):
            s = pl.ds(g * _L, _L)
            rows = lax.iota(jnp.int32, _L) + g * _L
            rv = r_v[s]
            acc = bi_v[s] + bj_v[s]
            for d in range(_D):
                col = jnp.full((_L,), d, jnp.int32)
                da = (plsc.load_gather(ai_v, [rows, col])
                      - plsc.load_gather(aj_v, [rows, col]))
                dv = (plsc.load_gather(vi_v, [rows, col])
                      - plsc.load_gather(vj_v, [rows, col]))
                dd = da + rv * dv
                acc = acc - dd * dd
            o_v[s] = acc
            return carry2

        lax.fori_loop(0, _G, grp_body, 0)
        pltpu.sync_copy(o_v, out_hbm.at[pl.ds(off, _E)])
        return carry

    lax.fori_loop(0, _NCHUNK, chunk_body, 0)


def kernel(times_list, node_pairs, x0, v, beta):
    a = _build_a(x0.reshape(1, _N * _D), v.reshape(_BINS, _N * _D))
    a_rows = a.reshape(_BINS * _N, _D)
    v_rows = v.reshape(_BINS * _N, _D)
    return _sc_intensity(times_list, node_pairs[0], node_pairs[1],
                         a_rows, v_rows, beta)


# speedup vs baseline: 56.0253x; 5.2117x over previous
import functools

import jax
import jax.numpy as jnp
from jax import lax
from jax.experimental import pallas as pl
from jax.experimental.pallas import tpu as pltpu
from jax.experimental.pallas import tpu_sc as plsc

_N = 100000
_D = 16
_BINS = 10
_P = 262144
_BIN_W = 1.0 / float(_BINS)

_NC, _NS, _L = 2, 16, 16
_NW = _NC * _NS
_EW = _P // _NW
_E = 1024
_NCHUNK = _EW // _E
_G = _E // _L

_mesh = plsc.VectorSubcoreMesh(
    core_axis_name="c", subcore_axis_name="s", num_cores=_NC, num_subcores=_NS
)

_CN = 625
_NCH_B = _N // (_NW * _CN)
_UNR = 5


@functools.partial(
    pl.kernel,
    mesh=_mesh,
    out_type=jax.ShapeDtypeStruct((_BINS * _N, _D), jnp.float32),
    compiler_params=pltpu.CompilerParams(
        needs_layout_passes=False, use_tc_tiling_on_sc=False
    ),
    scratch_types=[
        pltpu.VMEM((_CN, _D), jnp.float32),
        pltpu.VMEM((_CN, _D), jnp.float32),
    ],
)
def _sc_build_a(x0_hbm, v_hbm, a_hbm, acc_v, vb_v):
    _w = jnp.float32(_BIN_W)
    wid = lax.axis_index("s") * _NC + lax.axis_index("c")

    def chunk_body(k, carry):
        n0 = (wid * _NCH_B + k) * _CN
        pltpu.sync_copy(x0_hbm.at[pl.ds(n0, _CN)], acc_v)
        for b in range(_BINS):
            pltpu.sync_copy(acc_v, a_hbm.at[pl.ds(b * _N + n0, _CN)])
            if b < _BINS - 1:
                pltpu.sync_copy(v_hbm.at[pl.ds(b * _N + n0, _CN)], vb_v)

                def upd(t, carry2):
                    for u in range(_UNR):
                        r = t * _UNR + u
                        acc_v[r, :] = acc_v[r, :] + _w * vb_v[r, :]
                    return carry2

                lax.fori_loop(0, _CN // _UNR, upd, 0)
        return carry

    lax.fori_loop(0, _NCH_B, chunk_body, 0)


@functools.partial(
    pl.kernel,
    mesh=_mesh,
    out_type=jax.ShapeDtypeStruct((_P,), jnp.float32),
    compiler_params=pltpu.CompilerParams(
        needs_layout_passes=False, use_tc_tiling_on_sc=False
    ),
    scratch_types=[
        pltpu.VMEM((_E,), jnp.float32),
        pltpu.VMEM((_E,), jnp.int32),
        pltpu.VMEM((_E,), jnp.int32),
        pltpu.VMEM((_E,), jnp.int32),
        pltpu.VMEM((_E,), jnp.int32),
        pltpu.VMEM((_E,), jnp.float32),
        pltpu.VMEM((_E, _D), jnp.float32),
        pltpu.VMEM((_E, _D), jnp.float32),
        pltpu.VMEM((_E, _D), jnp.float32),
        pltpu.VMEM((_E, _D), jnp.float32),
        pltpu.VMEM((_E,), jnp.float32),
        pltpu.VMEM((_E,), jnp.float32),
        pltpu.VMEM((_E,), jnp.float32),
        pltpu.SemaphoreType.DMA,
    ],
)
def _sc_intensity(t_hbm, i_hbm, j_hbm, a_hbm, v_hbm, beta_hbm, out_hbm,
                  t_v, i_v, j_v, gi_v, gj_v, r_v,
                  ai_v, aj_v, vi_v, vj_v, bi_v, bj_v, o_v, sem):
    wid = lax.axis_index("s") * _NC + lax.axis_index("c")
    base = wid * _EW

    def chunk_body(c, carry):
        off = base + c * _E
        pltpu.sync_copy(t_hbm.at[pl.ds(off, _E)], t_v)
        pltpu.sync_copy(i_hbm.at[pl.ds(off, _E)], i_v)
        pltpu.sync_copy(j_hbm.at[pl.ds(off, _E)], j_v)

        def idx_body(g, carry2):
            s = pl.ds(g * _L, _L)
            t = t_v[s]
            b = (t / jnp.float32(_BIN_W)).astype(jnp.int32)
            b = jnp.where(b == _BINS, _BINS - 1, b)
            gi_v[s] = b * _N + i_v[s]
            gj_v[s] = b * _N + j_v[s]
            r_v[s] = jnp.remainder(t, jnp.float32(_BIN_W))
            return carry2

        lax.fori_loop(0, _G, idx_body, 0)

        cps = [
            pltpu.async_copy(a_hbm.at[gi_v], ai_v, sem),
            pltpu.async_copy(a_hbm.at[gj_v], aj_v, sem),
            pltpu.async_copy(v_hbm.at[gi_v], vi_v, sem),
            pltpu.async_copy(v_hbm.at[gj_v], vj_v, sem),
            pltpu.async_copy(beta_hbm.at[i_v], bi_v, sem),
            pltpu.async_copy(beta_hbm.at[j_v], bj_v, sem),
        ]
        for cp in cps:
            cp.wait()

        def grp_body(g, carry2):
            s = pl.ds(g * _L, _L)
            rows = lax.iota(jnp.int32, _L) + g * _L
            rv = r_v[s]
            acc = bi_v[s] + bj_v[s]
            for d in range(_D):
                col = jnp.full((_L,), d, jnp.int32)
                da = (plsc.load_gather(ai_v, [rows, col])
                      - plsc.load_gather(aj_v, [rows, col]))
                dv = (plsc.load_gather(vi_v, [rows, col])
                      - plsc.load_gather(vj_v, [rows, col]))
                dd = da + rv * dv
                acc = acc - dd * dd
            o_v[s] = acc
            return carry2

        lax.fori_loop(0, _G, grp_body, 0)
        pltpu.sync_copy(o_v, out_hbm.at[pl.ds(off, _E)])
        return carry

    lax.fori_loop(0, _NCHUNK, chunk_body, 0)


def kernel(times_list, node_pairs, x0, v, beta):
    v_rows = v.reshape(_BINS * _N, _D)
    a_rows = _sc_build_a(x0, v_rows)
    return _sc_intensity(times_list, node_pairs[0], node_pairs[1],
                         a_rows, v_rows, beta)


# speedup vs baseline: 56.5201x; 1.0088x over previous
import functools

import jax
import jax.numpy as jnp
from jax import lax
from jax.experimental import pallas as pl
from jax.experimental.pallas import tpu as pltpu
from jax.experimental.pallas import tpu_sc as plsc

_N = 100000
_D = 16
_BINS = 10
_P = 262144
_BIN_W = 1.0 / float(_BINS)

_NC, _NS, _L = 2, 16, 16
_NW = _NC * _NS
_EW = _P // _NW
_E = 512
_NCHUNK = _EW // _E
_G = _E // _L

_mesh = plsc.VectorSubcoreMesh(
    core_axis_name="c", subcore_axis_name="s", num_cores=_NC, num_subcores=_NS
)

_CN = 625
_NCH_B = _N // (_NW * _CN)
_UNR = 5


@functools.partial(
    pl.kernel,
    mesh=_mesh,
    out_type=(
        jax.ShapeDtypeStruct((_BINS * _N, _D), jnp.float32),
        jax.ShapeDtypeStruct((_BINS * _N, _D), jnp.float32),
    ),
    compiler_params=pltpu.CompilerParams(
        needs_layout_passes=False, use_tc_tiling_on_sc=False
    ),
    scratch_types=[
        pltpu.VMEM((_CN, _D), jnp.float32),
        pltpu.VMEM((_CN, _D), jnp.float32),
    ],
)
def _sc_build_a(x0_hbm, v_hbm, a_hbm, vlin_hbm, acc_v, vb_v):
    _w = jnp.float32(_BIN_W)
    wid = lax.axis_index("s") * _NC + lax.axis_index("c")

    def chunk_body(k, carry):
        n0 = (wid * _NCH_B + k) * _CN
        pltpu.sync_copy(x0_hbm.at[pl.ds(n0, _CN)], acc_v)
        for b in range(_BINS):
            pltpu.sync_copy(acc_v, a_hbm.at[pl.ds(b * _N + n0, _CN)])
            pltpu.sync_copy(v_hbm.at[b, pl.ds(n0, _CN), :], vb_v)
            pltpu.sync_copy(vb_v, vlin_hbm.at[pl.ds(b * _N + n0, _CN)])
            if b < _BINS - 1:

                def upd(t, carry2):
                    for u in range(_UNR):
                        r = t * _UNR + u
                        acc_v[r, :] = acc_v[r, :] + _w * vb_v[r, :]
                    return carry2

                lax.fori_loop(0, _CN // _UNR, upd, 0)
        return carry

    lax.fori_loop(0, _NCH_B, chunk_body, 0)


@functools.partial(
    pl.kernel,
    mesh=_mesh,
    out_type=jax.ShapeDtypeStruct((_P,), jnp.float32),
    compiler_params=pltpu.CompilerParams(
        needs_layout_passes=False, use_tc_tiling_on_sc=False
    ),
    scratch_types=[
        pltpu.VMEM((2, _E), jnp.float32),
        pltpu.VMEM((2, _E), jnp.int32),
        pltpu.VMEM((2, _E), jnp.int32),
        pltpu.VMEM((2, _E), jnp.int32),
        pltpu.VMEM((2, _E), jnp.int32),
        pltpu.VMEM((2, _E), jnp.float32),
        pltpu.VMEM((2, _E, _D), jnp.float32),
        pltpu.VMEM((2, _E, _D), jnp.float32),
        pltpu.VMEM((2, _E, _D), jnp.float32),
        pltpu.VMEM((2, _E, _D), jnp.float32),
        pltpu.VMEM((2, _E), jnp.float32),
        pltpu.VMEM((2, _E), jnp.float32),
        pltpu.VMEM((2, _E), jnp.float32),
        pltpu.SemaphoreType.DMA,
        pltpu.SemaphoreType.DMA,
    ],
)
def _sc_intensity(t_hbm, i_hbm, j_hbm, a_hbm, v_hbm, beta_hbm, out_hbm,
                  t_v, i_v, j_v, gi_v, gj_v, r_v,
                  ai_v, aj_v, vi_v, vj_v, bi_v, bj_v, o_v, sem0, sem1):
    wid = lax.axis_index("s") * _NC + lax.axis_index("c")
    base = wid * _EW
    sems = (sem0, sem1)

    def stage(c, s_):
        off = base + c * _E
        tb, ib, jb = t_v.at[s_], i_v.at[s_], j_v.at[s_]
        gib, gjb = gi_v.at[s_], gj_v.at[s_]
        pltpu.sync_copy(t_hbm.at[pl.ds(off, _E)], tb)
        pltpu.sync_copy(i_hbm.at[pl.ds(off, _E)], ib)
        pltpu.sync_copy(j_hbm.at[pl.ds(off, _E)], jb)

        def idx_body(g, carry2):
            sl = pl.ds(g * _L, _L)
            t = tb[sl]
            b = (t / jnp.float32(_BIN_W)).astype(jnp.int32)
            b = jnp.where(b == _BINS, _BINS - 1, b)
            gib[sl] = b * _N + ib[sl]
            gjb[sl] = b * _N + jb[sl]
            r_v.at[s_][sl] = jnp.remainder(t, jnp.float32(_BIN_W))
            return carry2

        lax.fori_loop(0, _G, idx_body, 0)
        sem = sems[s_]
        return [
            pltpu.async_copy(a_hbm.at[gib], ai_v.at[s_], sem),
            pltpu.async_copy(a_hbm.at[gjb], aj_v.at[s_], sem),
            pltpu.async_copy(v_hbm.at[gib], vi_v.at[s_], sem),
            pltpu.async_copy(v_hbm.at[gjb], vj_v.at[s_], sem),
            pltpu.async_copy(beta_hbm.at[ib], bi_v.at[s_], sem),
            pltpu.async_copy(beta_hbm.at[jb], bj_v.at[s_], sem),
        ]

    def compute(c, s_, cps):
        for cp in cps:
            cp.wait()
        off = base + c * _E
        aib, ajb, vib, vjb = ai_v.at[s_], aj_v.at[s_], vi_v.at[s_], vj_v.at[s_]
        ob = o_v.at[s_]

        def grp_body(g, carry2):
            sl = pl.ds(g * _L, _L)
            rows = lax.iota(jnp.int32, _L) + g * _L
            rv = r_v.at[s_][sl]
            acc = bi_v.at[s_][sl] + bj_v.at[s_][sl]
            for d in range(_D):
                col = jnp.full((_L,), d, jnp.int32)
                da = (plsc.load_gather(aib, [rows, col])
                      - plsc.load_gather(ajb, [rows, col]))
                dv = (plsc.load_gather(vib, [rows, col])
                      - plsc.load_gather(vjb, [rows, col]))
                dd = da + rv * dv
                acc = acc - dd * dd
            ob[sl] = acc
            return carry2

        lax.fori_loop(0, _G, grp_body, 0)
        pltpu.sync_copy(ob, out_hbm.at[pl.ds(off, _E)])

    cps = stage(0, 0)
    for c in range(1, _NCHUNK):
        s_ = c & 1
        nxt = stage(c, s_)
        compute(c - 1, 1 - s_, cps)
        cps = nxt
    compute(_NCHUNK - 1, (_NCHUNK - 1) & 1, cps)


def kernel(times_list, node_pairs, x0, v, beta):
    a_rows, v_rows = _sc_build_a(x0, v)
    return _sc_intensity(times_list, node_pairs[0], node_pairs[1],
                         a_rows, v_rows, beta)
